# baseline (device time: 193606 ns/iter reference)
import jax
import jax.numpy as jnp
from jax import lax
from jax.experimental import pallas as pl
from jax.experimental.pallas import tpu as pltpu

M = 8192
HALF = M // 2
HALF2 = HALF // 2
D = 4096
BLK = 256
N_CH = HALF2 // BLK
N_XD = 5
N_SW = 3


def kernel(partial, gamma):
    x2d = partial.reshape(M, D)
    g2d = gamma.reshape(1, D)

    def body(x_ref, g_ref, out_ref, contrib_ref, ld_ref, sb_ref, a_ref,
             b_ref, ob_ref, zsend, zrecv, swsend, swrecv, xsend, xrecv,
             fwsend, fwrecv, ostore, cp_sems, asem, bsem):
        my_x = lax.axis_index("x")
        my_y = lax.axis_index("y")
        my_z = lax.axis_index("z")
        znbr = (my_x, my_y, 1 - my_z)
        xnbr = (1 - my_x, my_y, my_z)
        ynbr = (my_x, 1 - my_y, my_z)

        barrier = pltpu.get_barrier_semaphore()
        for nbr in (znbr, xnbr, ynbr):
            pl.semaphore_signal(barrier, inc=1, device_id=nbr,
                                device_id_type=pl.DeviceIdType.MESH)
        pl.semaphore_wait(barrier, 3)

        theirs = (1 - my_z) * HALF
        mine = my_z * HALF
        part = my_x * HALF2
        partner = (1 - my_x) * HALF2

        def a_of(k):
            return my_y * 5 + k if k < N_SW else k

        def s_of(k):
            return (1 - my_y) * 5 + k

        def prep_zsend(k):
            slot = k % 2
            ac = a_of(k)
            cp = pltpu.make_async_copy(
                x_ref.at[pl.ds(theirs + part + ac * BLK, BLK), :], ld_ref,
                cp_sems.at[0])
            cp.start()
            cp.wait()
            sb_ref[slot] = ld_ref[...].astype(jnp.bfloat16)
            op = pltpu.make_async_remote_copy(
                src_ref=sb_ref.at[slot],
                dst_ref=contrib_ref.at[pl.ds(ac * BLK, BLK), :],
                send_sem=zsend.at[k],
                recv_sem=zrecv.at[k],
                device_id=znbr,
                device_id_type=pl.DeviceIdType.MESH,
            )
            op.start()
            return op

        def swap_send(k):
            sl = pl.ds(a_of(k) * BLK, BLK)
            op = pltpu.make_async_remote_copy(
                src_ref=contrib_ref.at[sl, :],
                dst_ref=contrib_ref.at[sl, :],
                send_sem=swsend.at[k],
                recv_sem=swrecv.at[k],
                device_id=ynbr,
                device_id_type=pl.DeviceIdType.MESH,
            )
            op.start()
            return op

        def forward(k):
            sl = pl.ds(partner + a_of(k) * BLK, BLK)
            op = pltpu.make_async_remote_copy(
                src_ref=out_ref.at[sl, :],
                dst_ref=out_ref.at[sl, :],
                send_sem=fwsend.at[k],
                recv_sem=fwrecv.at[k],
                device_id=ynbr,
                device_id_type=pl.DeviceIdType.MESH,
            )
            op.start()
            return op

        def issue_loads(c, ac):
            s = c % 2
            la = pltpu.make_async_copy(
                x_ref.at[pl.ds(mine + part + ac * BLK, BLK), :],
                a_ref.at[s], asem.at[s])
            lb = pltpu.make_async_copy(
                contrib_ref.at[pl.ds(ac * BLK, BLK), :], b_ref.at[s],
                bsem.at[s])
            la.start()
            lb.start()
            return (la, lb)

        users = {}

        def compute_chunk(c, ac, k_x, loads):
            loads[0].wait()
            loads[1].wait()
            s = c % 2
            y = a_ref[s] + b_ref[s].astype(jnp.float32)
            rms = jnp.sqrt(jnp.mean(y * y, axis=-1, keepdims=True) + 1e-6)
            if c >= 2:
                for kind, op in users[c - 2]:
                    op.wait() if kind == "l" else op.wait_send()
            ob_ref[s] = (y / rms * g_ref[...]).astype(jnp.bfloat16)
            o_op = pltpu.make_async_copy(
                ob_ref.at[s], out_ref.at[pl.ds(part + ac * BLK, BLK), :],
                ostore.at[s])
            o_op.start()
            users[c] = [("l", o_op)]
            if k_x is not None:
                x_op = pltpu.make_async_remote_copy(
                    src_ref=ob_ref.at[s],
                    dst_ref=out_ref.at[pl.ds(part + ac * BLK, BLK), :],
                    send_sem=xsend.at[k_x],
                    recv_sem=xrecv.at[k_x],
                    device_id=xnbr,
                    device_id_type=pl.DeviceIdType.MESH,
                )
                x_op.start()
                users[c].append(("s", x_op))
                return x_op
            return None

        zops = {0: prep_zsend(0)}
        zops[1] = prep_zsend(1)
        xops = {}
        swops = {}
        fwops = {}
        loads = {}
        zops[0].wait_recv()
        swops[0] = swap_send(0)
        loads[0] = issue_loads(0, a_of(0))
        for k in range(N_XD):
            if k + 2 < N_XD:
                zops[k].wait_send()
                zops[k + 2] = prep_zsend(k + 2)
            if k + 1 < N_XD:
                zops[k + 1].wait_recv()
                if k + 1 < N_SW:
                    swops[k + 1] = swap_send(k + 1)
                loads[k + 1] = issue_loads(k + 1, a_of(k + 1))
            else:
                swops[0].wait_recv()
                loads[N_XD] = issue_loads(N_XD, s_of(0))
            xops[k] = compute_chunk(k, a_of(k), k, loads[k])
            if k >= 1:
                xops[k - 1].wait_recv()
                if k - 1 < N_SW:
                    fwops[k - 1] = forward(k - 1)

        for j in range(N_SW):
            c = N_XD + j
            if j + 1 < N_SW:
                swops[j + 1].wait_recv()
                loads[c + 1] = issue_loads(c + 1, s_of(j + 1))
            compute_chunk(c, s_of(j), None, loads[c])
        xops[N_XD - 1].wait_recv()

        zops[N_XD - 2].wait_send()
        zops[N_XD - 1].wait_send()
        for k in range(N_SW):
            swops[k].wait_send()
            fwops[k].wait_send()
        for k in range(N_SW):
            fwops[k].wait_recv()
        for c in (N_CH - 2, N_CH - 1):
            for kind, op in users[c]:
                op.wait() if kind == "l" else op.wait_send()

    out, _ = pl.pallas_call(
        body,
        out_shape=(
            jax.ShapeDtypeStruct((HALF, D), jnp.bfloat16),
            jax.ShapeDtypeStruct((HALF2, D), jnp.bfloat16),
        ),
        in_specs=[
            pl.BlockSpec(memory_space=pl.ANY),
            pl.BlockSpec(memory_space=pltpu.VMEM),
        ],
        out_specs=(
            pl.BlockSpec(memory_space=pl.ANY),
            pl.BlockSpec(memory_space=pl.ANY),
        ),
        scratch_shapes=[
            pltpu.VMEM((BLK, D), jnp.float32),
            pltpu.VMEM((2, BLK, D), jnp.bfloat16),
            pltpu.VMEM((2, BLK, D), jnp.float32),
            pltpu.VMEM((2, BLK, D), jnp.bfloat16),
            pltpu.VMEM((2, BLK, D), jnp.bfloat16),
            pltpu.SemaphoreType.DMA((N_XD,)),
            pltpu.SemaphoreType.DMA((N_XD,)),
            pltpu.SemaphoreType.DMA((N_SW,)),
            pltpu.SemaphoreType.DMA((N_SW,)),
            pltpu.SemaphoreType.DMA((N_XD,)),
            pltpu.SemaphoreType.DMA((N_XD,)),
            pltpu.SemaphoreType.DMA((N_SW,)),
            pltpu.SemaphoreType.DMA((N_SW,)),
            pltpu.SemaphoreType.DMA((2,)),
            pltpu.SemaphoreType.DMA((1,)),
            pltpu.SemaphoreType.DMA((2,)),
            pltpu.SemaphoreType.DMA((2,)),
        ],
        compiler_params=pltpu.CompilerParams(collective_id=0),
    )(x2d, g2d)
    return out
